# baseline (speedup 1.0000x reference)
import jax
import jax.numpy as jnp
from jax.experimental import pallas as pl
from jax.experimental.pallas import tpu as pltpu

BLOCK_M = 2048
BLOCK_K = 512
NEG_INF = float("-inf")


def _router_kernel(x_ref, w_ref, b_ref, gates_ref, logits_ref, acc_ref, *, nk):
    kk = pl.program_id(1)

    part = jnp.dot(x_ref[...], w_ref[...], preferred_element_type=jnp.float32)

    @pl.when(kk == 0)
    def _init():
        acc_ref[...] = part + b_ref[...]

    @pl.when(kk > 0)
    def _acc():
        acc_ref[...] += part

    @pl.when(kk == nk - 1)
    def _route():
        logits = acc_ref[...]
        logits_ref[...] = logits

        col = jax.lax.broadcasted_iota(jnp.int32, logits.shape, 1)
        v1 = jnp.max(logits, axis=-1, keepdims=True)
        i1 = jnp.argmax(logits, axis=-1)[:, None]
        hit1 = col == i1
        masked = jnp.where(hit1, NEG_INF, logits)
        v2 = jnp.max(masked, axis=-1, keepdims=True)
        i2 = jnp.argmax(masked, axis=-1)[:, None]

        d = v1 - v2
        p1 = jax.lax.logistic(d)
        p2 = jax.lax.logistic(-d)

        gates_ref[...] = jnp.where(hit1, p1, jnp.where(col == i2, p2, 0.0))


@jax.jit
def kernel(x, gate_w, gate_b):
    import functools

    m, k = x.shape
    n = gate_w.shape[1]
    nk = k // BLOCK_K
    grid = (m // BLOCK_M, nk)
    gates, logits = pl.pallas_call(
        functools.partial(_router_kernel, nk=nk),
        grid=grid,
        in_specs=[
            pl.BlockSpec((BLOCK_M, BLOCK_K), lambda i, kk: (i, kk)),
            pl.BlockSpec((BLOCK_K, n), lambda i, kk: (kk, 0)),
            pl.BlockSpec((1, n), lambda i, kk: (0, 0)),
        ],
        out_specs=[
            pl.BlockSpec((BLOCK_M, n), lambda i, kk: (i, 0)),
            pl.BlockSpec((BLOCK_M, n), lambda i, kk: (i, 0)),
        ],
        out_shape=[
            jax.ShapeDtypeStruct((m, n), jnp.float32),
            jax.ShapeDtypeStruct((m, n), jnp.float32),
        ],
        scratch_shapes=[pltpu.VMEM((BLOCK_M, n), jnp.float32)],
        compiler_params=pltpu.CompilerParams(
            dimension_semantics=("parallel", "arbitrary"),
        ),
    )(x, gate_w, gate_b.reshape(1, n))
    return (gates, logits)


# speedup vs baseline: 1.3012x; 1.3012x over previous
import jax
import jax.numpy as jnp
from jax.experimental import pallas as pl
from jax.experimental.pallas import tpu as pltpu

BLOCK_M = 2048
NEG_INF = float("-inf")


def _router_kernel(x_ref, w_ref, b_ref, gates_ref, logits_ref):
    logits = (
        jnp.dot(x_ref[...], w_ref[...], preferred_element_type=jnp.float32)
        + b_ref[...]
    )
    logits_ref[...] = logits

    col = jax.lax.broadcasted_iota(jnp.int32, logits.shape, 1)

    v1 = jnp.max(logits, axis=-1, keepdims=True)
    i1 = jnp.argmax(logits, axis=-1)[:, None]
    hit1 = col == i1
    masked = jnp.where(hit1, NEG_INF, logits)
    v2 = jnp.max(masked, axis=-1, keepdims=True)
    i2 = jnp.argmax(masked, axis=-1)[:, None]

    d = v1 - v2
    p1 = jax.lax.logistic(d)
    p2 = jax.lax.logistic(-d)

    gates_ref[...] = jnp.where(hit1, p1, jnp.where(col == i2, p2, 0.0))


@jax.jit
def kernel(x, gate_w, gate_b):
    m, k = x.shape
    n = gate_w.shape[1]
    grid = (m // BLOCK_M,)
    gates, logits = pl.pallas_call(
        _router_kernel,
        grid=grid,
        in_specs=[
            pl.BlockSpec((BLOCK_M, k), lambda i: (i, 0)),
            pl.BlockSpec((k, n), lambda i: (0, 0)),
            pl.BlockSpec((1, n), lambda i: (0, 0)),
        ],
        out_specs=[
            pl.BlockSpec((BLOCK_M, n), lambda i: (i, 0)),
            pl.BlockSpec((BLOCK_M, n), lambda i: (i, 0)),
        ],
        out_shape=[
            jax.ShapeDtypeStruct((m, n), jnp.float32),
            jax.ShapeDtypeStruct((m, n), jnp.float32),
        ],
        compiler_params=pltpu.CompilerParams(
            dimension_semantics=("parallel",),
        ),
    )(x, gate_w, gate_b.reshape(1, n))
    return (gates, logits)
